# baseline (speedup 1.0000x reference)
import functools

import jax
import jax.numpy as jnp
from jax import lax
from jax.experimental import pallas as pl
from jax.experimental.pallas import tpu as pltpu
from jax.experimental.pallas import tpu_sc as plsc

NT = 1000000
BATCH = 16384

_info = plsc.get_sparse_core_info()
_NC, _NS = 1, _info.num_subcores
_NW = _NC * _NS
_PER_W = BATCH // _NW
_CHUNK = 128
_N_CHUNK = _PER_W // _CHUNK

_mesh = plsc.VectorSubcoreMesh(
    core_axis_name="c", subcore_axis_name="s", num_cores=1
)


@functools.partial(
    pl.kernel,
    out_type=jax.ShapeDtypeStruct((1, BATCH), jnp.float32),
    mesh=_mesh,
    scratch_types=[
        pltpu.VMEM((_PER_W,), jnp.int32),
        pltpu.VMEM((_PER_W,), jnp.float32),
        pltpu.SemaphoreType.DMA((_N_CHUNK,)),
        pltpu.SemaphoreType.DMA,
    ],
)
def _gather_kernel(times_t_hbm, perm_hbm, out_hbm, idx_v, vals_v, isem, gsem):
    wid = lax.axis_index("s") * _NC + lax.axis_index("c")
    base = wid * _PER_W
    idx_copies = [
        pltpu.async_copy(
            perm_hbm.at[pl.ds(base + j * _CHUNK, _CHUNK)],
            idx_v.at[pl.ds(j * _CHUNK, _CHUNK)],
            isem.at[j],
        )
        for j in range(_N_CHUNK)
    ]
    flat = times_t_hbm.at[0]
    copies = []
    for j in range(_N_CHUNK):
        idx_copies[j].wait()
        copies.append(
            pltpu.async_copy(
                flat.at[idx_v.at[pl.ds(j * _CHUNK, _CHUNK)]],
                vals_v.at[pl.ds(j * _CHUNK, _CHUNK)],
                gsem,
            )
        )
    for c in copies:
        c.wait()
    pltpu.sync_copy(vals_v, out_hbm.at[0].at[pl.ds(base, _PER_W)])


def kernel(times, perm):
    return _gather_kernel(times.T, perm.astype(jnp.int32)).T


# speedup vs baseline: 1.0058x; 1.0058x over previous
import functools

import jax
import jax.numpy as jnp
from jax import lax
from jax.experimental import pallas as pl
from jax.experimental.pallas import tpu as pltpu
from jax.experimental.pallas import tpu_sc as plsc

NT = 1000000
BATCH = 16384

_info = plsc.get_sparse_core_info()
_NC, _NS = 1, _info.num_subcores
_NW = _NC * _NS
_PER_W = BATCH // _NW
_CHUNK = 128
_N_CHUNK = _PER_W // _CHUNK

_mesh = plsc.VectorSubcoreMesh(
    core_axis_name="c", subcore_axis_name="s", num_cores=1
)


@functools.partial(
    pl.kernel,
    out_type=jax.ShapeDtypeStruct((1, BATCH), jnp.float32),
    mesh=_mesh,
    scratch_types=[
        pltpu.VMEM((_PER_W,), jnp.int32),
        pltpu.VMEM((_PER_W,), jnp.float32),
        pltpu.SemaphoreType.DMA,
    ],
)
def _gather_kernel(times_t_hbm, perm_hbm, out_hbm, idx_v, vals_v, sem):
    wid = lax.axis_index("s") * _NC + lax.axis_index("c")
    base = wid * _PER_W
    pltpu.sync_copy(perm_hbm.at[pl.ds(base, _PER_W)], idx_v)
    flat = times_t_hbm.at[0]
    copies = [
        pltpu.async_copy(
            flat.at[idx_v.at[pl.ds(j * _CHUNK, _CHUNK)]],
            vals_v.at[pl.ds(j * _CHUNK, _CHUNK)],
            sem,
        )
        for j in range(_N_CHUNK)
    ]
    for c in copies:
        c.wait()
    pltpu.sync_copy(vals_v, out_hbm.at[0].at[pl.ds(base, _PER_W)])


def kernel(times, perm):
    return _gather_kernel(times.T, perm.astype(jnp.int32)).T
